# baseline (speedup 1.0000x reference)
import functools
import math

import jax
import jax.numpy as jnp
from jax import lax
from jax.experimental import pallas as pl
from jax.experimental.pallas import tpu as pltpu
from jax.experimental.pallas import tpu_sc as plsc

N, E, D, B, R, VT = 10000, 320000, 128, 16, 10000, 3
BLK = 2000
GRID = N // BLK
INV_SQRT_D = 1.0 / math.sqrt(float(D))

NUM_WORKERS = 32
EDGES_PER_WORKER = E // NUM_WORKERS
CHUNK = 80
STAGE = 2000
NSTAGE = EDGES_PER_WORKER // STAGE
CHUNKS_PER_STAGE = STAGE // CHUNK
NPAD = 10112
ROWS_PER_TILE = NPAD // 16


def _tc1_body(batch_ref, ent_ref, rel_ref, q_ref, w1_ref, w2_ref, w3_ref,
              wm_ref, inj_e_ref, y1_ref, y2_ref):
    bcol = batch_ref[...]
    onehot = (bcol == lax.broadcasted_iota(jnp.int32, (BLK, B), 1)
              ).astype(jnp.float32)
    q = jnp.dot(onehot, q_ref[...], preferred_element_type=jnp.float32)
    ent = ent_ref[...]
    pre = (jnp.dot(ent, w1_ref[...], preferred_element_type=jnp.float32)
           + jnp.dot(q, w2_ref[...], preferred_element_type=jnp.float32))
    inj_e = ent + jnp.maximum(pre, 0.0)
    inj_e_ref[...] = inj_e
    y1_ref[...] = jnp.dot(inj_e, wm_ref[...], preferred_element_type=jnp.float32)
    rel = rel_ref[...]
    inj_r = rel + jnp.maximum(
        jnp.dot(rel, w3_ref[...], preferred_element_type=jnp.float32), 0.0)
    y2_ref[...] = jnp.dot(inj_r, wm_ref[...], preferred_element_type=jnp.float32)


def _sc_edge_body(y1_hbm, y2_hbm, src_hbm, rel_hbm, dst_hbm, z_hbm, out_hbm,
                  acc, src_all, rel_all, dst_all, dst_v,
                  ra0, rb0, ra1, rb1, sa0, sb0, sa1, sb1):
    c = lax.axis_index("c")
    s = lax.axis_index("s")
    w = s * 2 + c
    row0 = s * ROWS_PER_TILE
    pltpu.sync_copy(z_hbm.at[pl.ds(row0, ROWS_PER_TILE)],
                    acc.at[pl.ds(row0, ROWS_PER_TILE)])
    plsc.subcore_barrier()
    base = w * EDGES_PER_WORKER

    def _issue(lo, ra, rb, sa, sb):
        pltpu.async_copy(y1_hbm.at[src_all.at[pl.ds(lo, CHUNK)]], ra, sa)
        pltpu.async_copy(y2_hbm.at[rel_all.at[pl.ds(lo, CHUNK)]], rb, sb)

    def _wait(ra, rb, sa, sb):
        pltpu.make_async_copy(y1_hbm.at[pl.ds(0, CHUNK)], ra, sa).wait()
        pltpu.make_async_copy(y1_hbm.at[pl.ds(0, CHUNK)], rb, sb).wait()

    def _process(lo, ra, rb):
        def _erow(k, carry2):
            for j in range(D // 16):
                sl = pl.ds(j * 16, 16)
                ra[k, sl] = jnp.maximum(ra[k, sl] + rb[k, sl], 0.0)
            return carry2
        lax.fori_loop(0, CHUNK, _erow, 0)
        for j in range(CHUNK // 16):
            dst_v[pl.ds(j * 16, 16)] = dst_all[pl.ds(lo + j * 16, 16)]
        pltpu.sync_copy(ra, acc.at[dst_v], add=True)

    def _stage(g, carry0):
        sbase = base + g * STAGE
        pltpu.sync_copy(src_hbm.at[pl.ds(sbase, STAGE)], src_all)
        pltpu.sync_copy(rel_hbm.at[pl.ds(sbase, STAGE)], rel_all)
        pltpu.sync_copy(dst_hbm.at[pl.ds(sbase, STAGE)], dst_all)
        _issue(0, ra0, rb0, sa0, sb0)

        def _pair(i2, carry):
            c0 = 2 * i2 * CHUNK
            _issue(c0 + CHUNK, ra1, rb1, sa1, sb1)
            _wait(ra0, rb0, sa0, sb0)
            _process(c0, ra0, rb0)
            _issue(c0 + 2 * CHUNK, ra0, rb0, sa0, sb0)
            _wait(ra1, rb1, sa1, sb1)
            _process(c0 + CHUNK, ra1, rb1)
            return carry
        lax.fori_loop(0, (CHUNKS_PER_STAGE - 1) // 2, _pair, 0)
        _wait(ra0, rb0, sa0, sb0)
        _process((CHUNKS_PER_STAGE - 1) * CHUNK, ra0, rb0)
        return carry0
    lax.fori_loop(0, NSTAGE, _stage, 0)
    plsc.subcore_barrier()
    pltpu.sync_copy(acc.at[pl.ds(row0, ROWS_PER_TILE)],
                    out_hbm.at[c, pl.ds(row0, ROWS_PER_TILE)])


@functools.cache
def _get_sc_edge():
    return pl.kernel(
        _sc_edge_body,
        out_type=jax.ShapeDtypeStruct((2, NPAD, D), jnp.float32),
        mesh=plsc.VectorSubcoreMesh(core_axis_name="c", subcore_axis_name="s"),
        scratch_types=[
            pltpu.VMEM_SHARED((NPAD, D), jnp.float32),
            pltpu.VMEM((STAGE,), jnp.int32),
            pltpu.VMEM((STAGE,), jnp.int32),
            pltpu.VMEM((STAGE,), jnp.int32),
            pltpu.VMEM((CHUNK,), jnp.int32),
            pltpu.VMEM((CHUNK, D), jnp.float32),
            pltpu.VMEM((CHUNK, D), jnp.float32),
            pltpu.VMEM((CHUNK, D), jnp.float32),
            pltpu.VMEM((CHUNK, D), jnp.float32),
        ] + [pltpu.SemaphoreType.DMA] * 4,
    )


def _tc2_body(batch_ref, inj_e_ref, agg_ref, wout_ref, g_ref, bln_ref,
              vtt_ref, pw_ref, pb_ref, outvt_ref, outn_ref,
              numer_ref, denom_ref):
    i = pl.program_id(0)

    @pl.when(i == 0)
    def _init():
        numer_ref[...] = jnp.zeros((VT, B, D), jnp.float32)
        denom_ref[...] = jnp.zeros((B, VT), jnp.float32)

    bcol = batch_ref[...]
    onehot = (bcol == lax.broadcasted_iota(jnp.int32, (BLK, B), 1)
              ).astype(jnp.float32)
    agg = agg_ref[0] + agg_ref[1]
    h = jnp.maximum(
        jnp.dot(agg, wout_ref[...], preferred_element_type=jnp.float32), 0.0)
    x = inj_e_ref[...] + h
    mu = jnp.mean(x, axis=1, keepdims=True)
    xc = x - mu
    var = jnp.mean(xc * xc, axis=1, keepdims=True)
    emb = xc * lax.rsqrt(var + 1e-5) * g_ref[...] + bln_ref[...]
    scores = jnp.dot(emb, vtt_ref[...],
                     preferred_element_type=jnp.float32) * INV_SQRT_D
    ex = jnp.exp(scores)
    denom_ref[...] += lax.dot_general(
        onehot, ex, (((0,), (0,)), ((), ())),
        preferred_element_type=jnp.float32)
    for t in range(VT):
        wt = onehot * ex[:, t:t + 1]
        numer_ref[t] += lax.dot_general(
            wt, emb, (((0,), (0,)), ((), ())),
            preferred_element_type=jnp.float32)

    @pl.when(i == GRID - 1)
    def _final():
        dall = denom_ref[...]
        total = jnp.zeros((B, 1), jnp.float32)
        for t in range(VT):
            dv = dall[:, t:t + 1]
            safe = jnp.where(dv > 0.0, dv, 1.0)
            ov = jnp.where(dv > 0.0, numer_ref[t] / safe, 0.0)
            outvt_ref[:, t, :] = ov
            total = total + jnp.sum(ov * pw_ref[t:t + 1, :], axis=1,
                                    keepdims=True)
        outn_ref[...] = jnp.maximum(total + pb_ref[...], 0.0)


def kernel(queries, entities, relations, x_coo, batch, inj_W1, inj_W2,
           inj_W3, enc_Wmsg, enc_Wout, enc_Wrel, ln_g, ln_b, vt_tokens,
           pred_W, pred_b):
    del enc_Wrel
    batch2 = batch.astype(jnp.int32).reshape(N, 1)
    src = x_coo[:, 0].astype(jnp.int32)
    rel = x_coo[:, 1].astype(jnp.int32)
    dst = x_coo[:, 2].astype(jnp.int32)

    full = lambda shape: pl.BlockSpec(shape, lambda i: (0,) * len(shape))
    inj_e, y1, y2 = pl.pallas_call(
        _tc1_body,
        grid=(GRID,),
        in_specs=[
            pl.BlockSpec((BLK, 1), lambda i: (i, 0)),
            pl.BlockSpec((BLK, D), lambda i: (i, 0)),
            pl.BlockSpec((BLK, D), lambda i: (i, 0)),
            full((B, D)), full((D, D)), full((D, D)), full((D, D)),
            full((D, D)),
        ],
        out_specs=[pl.BlockSpec((BLK, D), lambda i: (i, 0))] * 3,
        out_shape=[jax.ShapeDtypeStruct((N, D), jnp.float32)] * 3,
    )(batch2, entities, relations, queries, inj_W1, inj_W2, inj_W3, enc_Wmsg)

    aggpair = _get_sc_edge()(y1, y2, src, rel, dst,
                             jnp.zeros((NPAD, D), jnp.float32))

    out_vt, out_n = pl.pallas_call(
        _tc2_body,
        grid=(GRID,),
        in_specs=[
            pl.BlockSpec((BLK, 1), lambda i: (i, 0)),
            pl.BlockSpec((BLK, D), lambda i: (i, 0)),
            pl.BlockSpec((2, BLK, D), lambda i: (0, i, 0)),
            full((D, D)), full((1, D)), full((1, D)), full((D, VT)),
            full((VT, D)), full((1, 1)),
        ],
        out_specs=[
            pl.BlockSpec((B, VT, D), lambda i: (0, 0, 0)),
            pl.BlockSpec((B, 1), lambda i: (0, 0)),
        ],
        out_shape=[
            jax.ShapeDtypeStruct((B, VT, D), jnp.float32),
            jax.ShapeDtypeStruct((B, 1), jnp.float32),
        ],
        scratch_shapes=[
            pltpu.VMEM((VT, B, D), jnp.float32),
            pltpu.VMEM((B, VT), jnp.float32),
        ],
    )(batch2, inj_e, aggpair, enc_Wout, ln_g.reshape(1, D),
      ln_b.reshape(1, D), vt_tokens.T, pred_W.reshape(VT, D),
      pred_b.reshape(1, 1))
    return out_vt, out_n


# speedup vs baseline: 1.0424x; 1.0424x over previous
import functools
import math

import jax
import jax.numpy as jnp
from jax import lax
from jax.experimental import pallas as pl
from jax.experimental.pallas import tpu as pltpu
from jax.experimental.pallas import tpu_sc as plsc

N, E, D, B, R, VT = 10000, 320000, 128, 16, 10000, 3
BLK = 2000
GRID = N // BLK
INV_SQRT_D = 1.0 / math.sqrt(float(D))

NUM_WORKERS = 32
EDGES_PER_WORKER = E // NUM_WORKERS
CHUNK = 80
STAGE = 2000
NSTAGE = EDGES_PER_WORKER // STAGE
CHUNKS_PER_STAGE = STAGE // CHUNK
NPAD = 10112
ROWS_PER_TILE = NPAD // 16


def _tc1_body(batch_ref, ent_ref, rel_ref, q_ref, w1_ref, w2_ref, w3_ref,
              wm_ref, inj_e_ref, y1_ref, y2_ref):
    bcol = batch_ref[...]
    onehot = (bcol == lax.broadcasted_iota(jnp.int32, (BLK, B), 1)
              ).astype(jnp.float32)
    q = jnp.dot(onehot, q_ref[...], preferred_element_type=jnp.float32)
    ent = ent_ref[...]
    pre = (jnp.dot(ent, w1_ref[...], preferred_element_type=jnp.float32)
           + jnp.dot(q, w2_ref[...], preferred_element_type=jnp.float32))
    inj_e = ent + jnp.maximum(pre, 0.0)
    inj_e_ref[...] = inj_e
    y1_ref[...] = jnp.dot(inj_e, wm_ref[...], preferred_element_type=jnp.float32)
    rel = rel_ref[...]
    inj_r = rel + jnp.maximum(
        jnp.dot(rel, w3_ref[...], preferred_element_type=jnp.float32), 0.0)
    y2_ref[...] = jnp.dot(inj_r, wm_ref[...], preferred_element_type=jnp.float32)


def _sc_edge_body(y1_hbm, y2_hbm, src_hbm, rel_hbm, dst_hbm, out_hbm,
                  acc, src_all, rel_all, dst_all, dst_v,
                  ra0, rb0, ra1, rb1, sa0, sb0, sa1, sb1):
    c = lax.axis_index("c")
    s = lax.axis_index("s")
    w = s * 2 + c
    def _zrow(i, carry):
        for j in range(D // 16):
            ra0[i, pl.ds(j * 16, 16)] = jnp.zeros((16,), jnp.float32)
        return carry
    lax.fori_loop(0, CHUNK, _zrow, 0)
    row0 = s * ROWS_PER_TILE
    _nz = ROWS_PER_TILE // CHUNK
    for k in range(_nz):
        pltpu.sync_copy(ra0, acc.at[pl.ds(row0 + k * CHUNK, CHUNK)])
    pltpu.sync_copy(ra0.at[pl.ds(0, ROWS_PER_TILE - _nz * CHUNK)],
                    acc.at[pl.ds(row0 + _nz * CHUNK,
                                 ROWS_PER_TILE - _nz * CHUNK)])
    plsc.subcore_barrier()
    base = w * EDGES_PER_WORKER

    def _issue(lo, ra, rb, sa, sb):
        pltpu.async_copy(y1_hbm.at[src_all.at[pl.ds(lo, CHUNK)]], ra, sa)
        pltpu.async_copy(y2_hbm.at[rel_all.at[pl.ds(lo, CHUNK)]], rb, sb)

    def _wait(ra, rb, sa, sb):
        pltpu.make_async_copy(y1_hbm.at[pl.ds(0, CHUNK)], ra, sa).wait()
        pltpu.make_async_copy(y1_hbm.at[pl.ds(0, CHUNK)], rb, sb).wait()

    def _process(lo, ra, rb):
        def _erow(k, carry2):
            for j in range(D // 16):
                sl = pl.ds(j * 16, 16)
                ra[k, sl] = jnp.maximum(ra[k, sl] + rb[k, sl], 0.0)
            return carry2
        lax.fori_loop(0, CHUNK, _erow, 0)
        for j in range(CHUNK // 16):
            dst_v[pl.ds(j * 16, 16)] = dst_all[pl.ds(lo + j * 16, 16)]
        pltpu.sync_copy(ra, acc.at[dst_v], add=True)

    def _stage(g, carry0):
        sbase = base + g * STAGE
        i0 = pltpu.async_copy(src_hbm.at[pl.ds(sbase, STAGE)], src_all, sa0)
        i1 = pltpu.async_copy(rel_hbm.at[pl.ds(sbase, STAGE)], rel_all, sb0)
        i2 = pltpu.async_copy(dst_hbm.at[pl.ds(sbase, STAGE)], dst_all, sa1)
        i0.wait()
        i1.wait()
        i2.wait()
        _issue(0, ra0, rb0, sa0, sb0)

        def _pair(i2, carry):
            c0 = 2 * i2 * CHUNK
            _issue(c0 + CHUNK, ra1, rb1, sa1, sb1)
            _wait(ra0, rb0, sa0, sb0)
            _process(c0, ra0, rb0)
            _issue(c0 + 2 * CHUNK, ra0, rb0, sa0, sb0)
            _wait(ra1, rb1, sa1, sb1)
            _process(c0 + CHUNK, ra1, rb1)
            return carry
        lax.fori_loop(0, (CHUNKS_PER_STAGE - 1) // 2, _pair, 0)
        _wait(ra0, rb0, sa0, sb0)
        _process((CHUNKS_PER_STAGE - 1) * CHUNK, ra0, rb0)
        return carry0
    lax.fori_loop(0, NSTAGE, _stage, 0)
    plsc.subcore_barrier()
    pltpu.sync_copy(acc.at[pl.ds(row0, ROWS_PER_TILE)],
                    out_hbm.at[c, pl.ds(row0, ROWS_PER_TILE)])


@functools.cache
def _get_sc_edge():
    return pl.kernel(
        _sc_edge_body,
        out_type=jax.ShapeDtypeStruct((2, NPAD, D), jnp.float32),
        mesh=plsc.VectorSubcoreMesh(core_axis_name="c", subcore_axis_name="s"),
        scratch_types=[
            pltpu.VMEM_SHARED((NPAD, D), jnp.float32),
            pltpu.VMEM((STAGE,), jnp.int32),
            pltpu.VMEM((STAGE,), jnp.int32),
            pltpu.VMEM((STAGE,), jnp.int32),
            pltpu.VMEM((CHUNK,), jnp.int32),
            pltpu.VMEM((CHUNK, D), jnp.float32),
            pltpu.VMEM((CHUNK, D), jnp.float32),
            pltpu.VMEM((CHUNK, D), jnp.float32),
            pltpu.VMEM((CHUNK, D), jnp.float32),
        ] + [pltpu.SemaphoreType.DMA] * 4,
    )


def _tc2_body(batch_ref, inj_e_ref, agg_ref, wout_ref, g_ref, bln_ref,
              vtt_ref, pw_ref, pb_ref, outvt_ref, outn_ref,
              numer_ref, denom_ref):
    i = pl.program_id(0)

    @pl.when(i == 0)
    def _init():
        numer_ref[...] = jnp.zeros((VT, B, D), jnp.float32)
        denom_ref[...] = jnp.zeros((B, VT), jnp.float32)

    bcol = batch_ref[...]
    onehot = (bcol == lax.broadcasted_iota(jnp.int32, (BLK, B), 1)
              ).astype(jnp.float32)
    agg = agg_ref[0] + agg_ref[1]
    h = jnp.maximum(
        jnp.dot(agg, wout_ref[...], preferred_element_type=jnp.float32), 0.0)
    x = inj_e_ref[...] + h
    mu = jnp.mean(x, axis=1, keepdims=True)
    xc = x - mu
    var = jnp.mean(xc * xc, axis=1, keepdims=True)
    emb = xc * lax.rsqrt(var + 1e-5) * g_ref[...] + bln_ref[...]
    scores = jnp.dot(emb, vtt_ref[...],
                     preferred_element_type=jnp.float32) * INV_SQRT_D
    ex = jnp.exp(scores)
    denom_ref[...] += lax.dot_general(
        onehot, ex, (((0,), (0,)), ((), ())),
        preferred_element_type=jnp.float32)
    for t in range(VT):
        wt = onehot * ex[:, t:t + 1]
        numer_ref[t] += lax.dot_general(
            wt, emb, (((0,), (0,)), ((), ())),
            preferred_element_type=jnp.float32)

    @pl.when(i == GRID - 1)
    def _final():
        dall = denom_ref[...]
        total = jnp.zeros((B, 1), jnp.float32)
        for t in range(VT):
            dv = dall[:, t:t + 1]
            safe = jnp.where(dv > 0.0, dv, 1.0)
            ov = jnp.where(dv > 0.0, numer_ref[t] / safe, 0.0)
            outvt_ref[:, t, :] = ov
            total = total + jnp.sum(ov * pw_ref[t:t + 1, :], axis=1,
                                    keepdims=True)
        outn_ref[...] = jnp.maximum(total + pb_ref[...], 0.0)


def kernel(queries, entities, relations, x_coo, batch, inj_W1, inj_W2,
           inj_W3, enc_Wmsg, enc_Wout, enc_Wrel, ln_g, ln_b, vt_tokens,
           pred_W, pred_b):
    del enc_Wrel
    batch2 = batch.astype(jnp.int32).reshape(N, 1)
    src = x_coo[:, 0].astype(jnp.int32)
    rel = x_coo[:, 1].astype(jnp.int32)
    dst = x_coo[:, 2].astype(jnp.int32)

    full = lambda shape: pl.BlockSpec(shape, lambda i: (0,) * len(shape))
    inj_e, y1, y2 = pl.pallas_call(
        _tc1_body,
        grid=(GRID,),
        in_specs=[
            pl.BlockSpec((BLK, 1), lambda i: (i, 0)),
            pl.BlockSpec((BLK, D), lambda i: (i, 0)),
            pl.BlockSpec((BLK, D), lambda i: (i, 0)),
            full((B, D)), full((D, D)), full((D, D)), full((D, D)),
            full((D, D)),
        ],
        out_specs=[pl.BlockSpec((BLK, D), lambda i: (i, 0))] * 3,
        out_shape=[jax.ShapeDtypeStruct((N, D), jnp.float32)] * 3,
    )(batch2, entities, relations, queries, inj_W1, inj_W2, inj_W3, enc_Wmsg)

    aggpair = _get_sc_edge()(y1, y2, src, rel, dst)

    out_vt, out_n = pl.pallas_call(
        _tc2_body,
        grid=(GRID,),
        in_specs=[
            pl.BlockSpec((BLK, 1), lambda i: (i, 0)),
            pl.BlockSpec((BLK, D), lambda i: (i, 0)),
            pl.BlockSpec((2, BLK, D), lambda i: (0, i, 0)),
            full((D, D)), full((1, D)), full((1, D)), full((D, VT)),
            full((VT, D)), full((1, 1)),
        ],
        out_specs=[
            pl.BlockSpec((B, VT, D), lambda i: (0, 0, 0)),
            pl.BlockSpec((B, 1), lambda i: (0, 0)),
        ],
        out_shape=[
            jax.ShapeDtypeStruct((B, VT, D), jnp.float32),
            jax.ShapeDtypeStruct((B, 1), jnp.float32),
        ],
        scratch_shapes=[
            pltpu.VMEM((VT, B, D), jnp.float32),
            pltpu.VMEM((B, VT), jnp.float32),
        ],
    )(batch2, inj_e, aggpair, enc_Wout, ln_g.reshape(1, D),
      ln_b.reshape(1, D), vt_tokens.T, pred_W.reshape(VT, D),
      pred_b.reshape(1, 1))
    return out_vt, out_n


# speedup vs baseline: 1.0465x; 1.0039x over previous
import functools
import math

import jax
import jax.numpy as jnp
from jax import lax
from jax.experimental import pallas as pl
from jax.experimental.pallas import tpu as pltpu
from jax.experimental.pallas import tpu_sc as plsc

N, E, D, B, R, VT = 10000, 320000, 128, 16, 10000, 3
BLK = 2000
GRID = N // BLK
INV_SQRT_D = 1.0 / math.sqrt(float(D))

NUM_WORKERS = 32
EDGES_PER_WORKER = E // NUM_WORKERS
CHUNK = 80
STAGE = 2000
NSTAGE = EDGES_PER_WORKER // STAGE
CHUNKS_PER_STAGE = STAGE // CHUNK
NPAD = 10112
ROWS_PER_TILE = NPAD // 16


def _tc1_body(batch_ref, ent_ref, rel_ref, q_ref, w1_ref, w2_ref, w3_ref,
              wm_ref, inj_e_ref, y1_ref, y2_ref):
    bcol = batch_ref[...]
    onehot = (bcol == lax.broadcasted_iota(jnp.int32, (BLK, B), 1)
              ).astype(jnp.float32)
    q = jnp.dot(onehot, q_ref[...], preferred_element_type=jnp.float32)
    ent = ent_ref[...]
    pre = (jnp.dot(ent, w1_ref[...], preferred_element_type=jnp.float32)
           + jnp.dot(q, w2_ref[...], preferred_element_type=jnp.float32))
    inj_e = ent + jnp.maximum(pre, 0.0)
    inj_e_ref[...] = inj_e
    y1_ref[...] = jnp.dot(inj_e, wm_ref[...], preferred_element_type=jnp.float32)
    rel = rel_ref[...]
    inj_r = rel + jnp.maximum(
        jnp.dot(rel, w3_ref[...], preferred_element_type=jnp.float32), 0.0)
    y2_ref[...] = jnp.dot(inj_r, wm_ref[...], preferred_element_type=jnp.float32)


def _sc_edge_body(y1_hbm, y2_hbm, src_hbm, rel_hbm, dst_hbm, out_hbm,
                  acc, src_all, rel_all, dst_all, dst_v,
                  ra0, rb0, ra1, rb1, sa0, sb0, sa1, sb1):
    c = lax.axis_index("c")
    s = lax.axis_index("s")
    w = s * 2 + c
    def _zrow(i, carry):
        for j in range(D // 16):
            ra0[i, pl.ds(j * 16, 16)] = jnp.zeros((16,), jnp.float32)
        return carry
    lax.fori_loop(0, CHUNK, _zrow, 0)
    row0 = s * ROWS_PER_TILE
    _nz = ROWS_PER_TILE // CHUNK
    for k in range(_nz):
        pltpu.sync_copy(ra0, acc.at[pl.ds(row0 + k * CHUNK, CHUNK)])
    pltpu.sync_copy(ra0.at[pl.ds(0, ROWS_PER_TILE - _nz * CHUNK)],
                    acc.at[pl.ds(row0 + _nz * CHUNK,
                                 ROWS_PER_TILE - _nz * CHUNK)])
    plsc.subcore_barrier()
    base = w * EDGES_PER_WORKER

    def _issue(lo, ra, rb, sa, sb):
        pltpu.async_copy(y1_hbm.at[src_all.at[pl.ds(lo, CHUNK)]], ra, sa)
        pltpu.async_copy(y2_hbm.at[rel_all.at[pl.ds(lo, CHUNK)]], rb, sb)

    def _wait(ra, rb, sa, sb):
        pltpu.make_async_copy(y1_hbm.at[pl.ds(0, CHUNK)], ra, sa).wait()
        pltpu.make_async_copy(y1_hbm.at[pl.ds(0, CHUNK)], rb, sb).wait()

    def _process(lo, ra, rb):
        def _erow(k, carry2):
            for r in range(2):
                kk = k * 2 + r
                for j in range(D // 16):
                    sl = pl.ds(j * 16, 16)
                    ra[kk, sl] = jnp.maximum(ra[kk, sl] + rb[kk, sl], 0.0)
            return carry2
        lax.fori_loop(0, CHUNK // 2, _erow, 0)
        for j in range(CHUNK // 16):
            dst_v[pl.ds(j * 16, 16)] = dst_all[pl.ds(lo + j * 16, 16)]
        pltpu.sync_copy(ra, acc.at[dst_v], add=True)

    def _stage(g, carry0):
        sbase = base + g * STAGE
        i0 = pltpu.async_copy(src_hbm.at[pl.ds(sbase, STAGE)], src_all, sa0)
        i1 = pltpu.async_copy(rel_hbm.at[pl.ds(sbase, STAGE)], rel_all, sb0)
        i2 = pltpu.async_copy(dst_hbm.at[pl.ds(sbase, STAGE)], dst_all, sa1)
        i0.wait()
        i1.wait()
        i2.wait()
        _issue(0, ra0, rb0, sa0, sb0)

        def _pair(i2, carry):
            c0 = 2 * i2 * CHUNK
            _issue(c0 + CHUNK, ra1, rb1, sa1, sb1)
            _wait(ra0, rb0, sa0, sb0)
            _process(c0, ra0, rb0)
            _issue(c0 + 2 * CHUNK, ra0, rb0, sa0, sb0)
            _wait(ra1, rb1, sa1, sb1)
            _process(c0 + CHUNK, ra1, rb1)
            return carry
        lax.fori_loop(0, (CHUNKS_PER_STAGE - 1) // 2, _pair, 0)
        _wait(ra0, rb0, sa0, sb0)
        _process((CHUNKS_PER_STAGE - 1) * CHUNK, ra0, rb0)
        return carry0
    lax.fori_loop(0, NSTAGE, _stage, 0)
    plsc.subcore_barrier()
    pltpu.sync_copy(acc.at[pl.ds(row0, ROWS_PER_TILE)],
                    out_hbm.at[c, pl.ds(row0, ROWS_PER_TILE)])


@functools.cache
def _get_sc_edge():
    return pl.kernel(
        _sc_edge_body,
        out_type=jax.ShapeDtypeStruct((2, NPAD, D), jnp.float32),
        mesh=plsc.VectorSubcoreMesh(core_axis_name="c", subcore_axis_name="s"),
        scratch_types=[
            pltpu.VMEM_SHARED((NPAD, D), jnp.float32),
            pltpu.VMEM((STAGE,), jnp.int32),
            pltpu.VMEM((STAGE,), jnp.int32),
            pltpu.VMEM((STAGE,), jnp.int32),
            pltpu.VMEM((CHUNK,), jnp.int32),
            pltpu.VMEM((CHUNK, D), jnp.float32),
            pltpu.VMEM((CHUNK, D), jnp.float32),
            pltpu.VMEM((CHUNK, D), jnp.float32),
            pltpu.VMEM((CHUNK, D), jnp.float32),
        ] + [pltpu.SemaphoreType.DMA] * 4,
    )


def _tc2_body(batch_ref, inj_e_ref, agg_ref, wout_ref, g_ref, bln_ref,
              vtt_ref, pw_ref, pb_ref, outvt_ref, outn_ref,
              numer_ref, denom_ref):
    i = pl.program_id(0)

    @pl.when(i == 0)
    def _init():
        numer_ref[...] = jnp.zeros((VT, B, D), jnp.float32)
        denom_ref[...] = jnp.zeros((B, VT), jnp.float32)

    bcol = batch_ref[...]
    onehot = (bcol == lax.broadcasted_iota(jnp.int32, (BLK, B), 1)
              ).astype(jnp.float32)
    agg = agg_ref[0] + agg_ref[1]
    h = jnp.maximum(
        jnp.dot(agg, wout_ref[...], preferred_element_type=jnp.float32), 0.0)
    x = inj_e_ref[...] + h
    mu = jnp.mean(x, axis=1, keepdims=True)
    xc = x - mu
    var = jnp.mean(xc * xc, axis=1, keepdims=True)
    emb = xc * lax.rsqrt(var + 1e-5) * g_ref[...] + bln_ref[...]
    scores = jnp.dot(emb, vtt_ref[...],
                     preferred_element_type=jnp.float32) * INV_SQRT_D
    ex = jnp.exp(scores)
    denom_ref[...] += lax.dot_general(
        onehot, ex, (((0,), (0,)), ((), ())),
        preferred_element_type=jnp.float32)
    for t in range(VT):
        wt = onehot * ex[:, t:t + 1]
        numer_ref[t] += lax.dot_general(
            wt, emb, (((0,), (0,)), ((), ())),
            preferred_element_type=jnp.float32)

    @pl.when(i == GRID - 1)
    def _final():
        dall = denom_ref[...]
        total = jnp.zeros((B, 1), jnp.float32)
        for t in range(VT):
            dv = dall[:, t:t + 1]
            safe = jnp.where(dv > 0.0, dv, 1.0)
            ov = jnp.where(dv > 0.0, numer_ref[t] / safe, 0.0)
            outvt_ref[:, t, :] = ov
            total = total + jnp.sum(ov * pw_ref[t:t + 1, :], axis=1,
                                    keepdims=True)
        outn_ref[...] = jnp.maximum(total + pb_ref[...], 0.0)


def kernel(queries, entities, relations, x_coo, batch, inj_W1, inj_W2,
           inj_W3, enc_Wmsg, enc_Wout, enc_Wrel, ln_g, ln_b, vt_tokens,
           pred_W, pred_b):
    del enc_Wrel
    batch2 = batch.astype(jnp.int32).reshape(N, 1)
    src = x_coo[:, 0].astype(jnp.int32)
    rel = x_coo[:, 1].astype(jnp.int32)
    dst = x_coo[:, 2].astype(jnp.int32)

    full = lambda shape: pl.BlockSpec(shape, lambda i: (0,) * len(shape))
    inj_e, y1, y2 = pl.pallas_call(
        _tc1_body,
        grid=(GRID,),
        in_specs=[
            pl.BlockSpec((BLK, 1), lambda i: (i, 0)),
            pl.BlockSpec((BLK, D), lambda i: (i, 0)),
            pl.BlockSpec((BLK, D), lambda i: (i, 0)),
            full((B, D)), full((D, D)), full((D, D)), full((D, D)),
            full((D, D)),
        ],
        out_specs=[pl.BlockSpec((BLK, D), lambda i: (i, 0))] * 3,
        out_shape=[jax.ShapeDtypeStruct((N, D), jnp.float32)] * 3,
    )(batch2, entities, relations, queries, inj_W1, inj_W2, inj_W3, enc_Wmsg)

    aggpair = _get_sc_edge()(y1, y2, src, rel, dst)

    out_vt, out_n = pl.pallas_call(
        _tc2_body,
        grid=(GRID,),
        in_specs=[
            pl.BlockSpec((BLK, 1), lambda i: (i, 0)),
            pl.BlockSpec((BLK, D), lambda i: (i, 0)),
            pl.BlockSpec((2, BLK, D), lambda i: (0, i, 0)),
            full((D, D)), full((1, D)), full((1, D)), full((D, VT)),
            full((VT, D)), full((1, 1)),
        ],
        out_specs=[
            pl.BlockSpec((B, VT, D), lambda i: (0, 0, 0)),
            pl.BlockSpec((B, 1), lambda i: (0, 0)),
        ],
        out_shape=[
            jax.ShapeDtypeStruct((B, VT, D), jnp.float32),
            jax.ShapeDtypeStruct((B, 1), jnp.float32),
        ],
        scratch_shapes=[
            pltpu.VMEM((VT, B, D), jnp.float32),
            pltpu.VMEM((B, VT), jnp.float32),
        ],
    )(batch2, inj_e, aggpair, enc_Wout, ln_g.reshape(1, D),
      ln_b.reshape(1, D), vt_tokens.T, pred_W.reshape(VT, D),
      pred_b.reshape(1, 1))
    return out_vt, out_n
